# baseline (speedup 1.0000x reference)
import functools

import jax
import jax.numpy as jnp
from jax import lax
from jax.experimental import pallas as pl
from jax.experimental.pallas import tpu as pltpu
from jax.experimental.pallas import tpu_sc as plsc

_NC, _NS, _L = 2, 16, 16
_NW = _NC * _NS
_C = 1048576
_NROWS = 12

_CW = _C // _NW
_CH = _CW // 2
_NQ = 2 * _NROWS
_NB = 4
assert _CH % 8 == 0 and _CH % _L == 0
_UNROLL = 8

_mesh = plsc.VectorSubcoreMesh(
    core_axis_name="c", subcore_axis_name="s",
    num_cores=_NC, num_subcores=_NS)

_ABS_MASK = 0x7FFFFFFF
_INF_BITS = 0x7F800000


def _row_addr(k):
    q = k // 4
    m = k % 4
    b = m + 1 + jnp.where((m >= 2) & (q > 0), 1, 0)
    return q, b


@functools.partial(
    pl.kernel,
    out_type=jax.ShapeDtypeStruct((_NW, _L), jnp.int32),
    mesh=_mesh,
    scratch_types=[
        pltpu.VMEM((_NB * _CH,), jnp.float32),
        pltpu.VMEM((_L,), jnp.int32),
        pltpu.SemaphoreType.DMA((_NB,)),
    ],
)
def _sc_masked_selfcmp(x_hbm, out_hbm, ring, accv, sems):
    cid = lax.axis_index("c")
    sid = lax.axis_index("s")
    wid = sid * _NC + cid
    colbase = wid * _CW

    absmask = jnp.full((_L,), _ABS_MASK, jnp.int32)

    def start(q):
        a, b = _row_addr(q >> 1)
        cb = colbase + (q & 1) * _CH
        slot = lax.rem(q, _NB)
        pltpu.async_copy(
            x_hbm.at[a, b, pl.ds(cb, _CH)],
            ring.at[pl.ds(slot * _CH, _CH)], sems.at[slot])

    for q in range(_NB - 1):
        start(jnp.int32(q))

    def step(q, acc):
        @pl.when(q + (_NB - 1) < _NQ)
        def _():
            start(q + (_NB - 1))

        slot = lax.rem(q, _NB)
        base0 = slot * _CH
        pltpu.make_async_copy(
            x_hbm.at[0, 1, pl.ds(colbase, _CH)],
            ring.at[pl.ds(base0, _CH)], sems.at[slot]).wait()

        def body(j, acc):
            base = base0 + j * (_L * _UNROLL)
            for u in range(_UNROLL):
                v = ring[pl.ds(base + u * _L, _L)]
                bits = lax.bitcast_convert_type(v, jnp.int32) & absmask
                acc = jnp.maximum(acc, bits)
            return acc
        return lax.fori_loop(0, _CH // (_L * _UNROLL), body, acc)

    acc = lax.fori_loop(0, _NQ, step, jnp.zeros((_L,), jnp.int32))

    accv[...] = acc
    pltpu.sync_copy(accv, out_hbm.at[wid])


def kernel(x):
    partials = _sc_masked_selfcmp(x)
    return jnp.max(partials) <= jnp.int32(_INF_BITS)
